# baseline (speedup 1.0000x reference)
import functools

import jax
import jax.numpy as jnp
from jax import lax
from jax.experimental import pallas as pl
from jax.experimental.pallas import tpu as pltpu
from jax.experimental.pallas import tpu_sc as plsc

EMB = 64
BATCH = 4096
SEQ = 200
B_TOTAL = BATCH * SEQ
NUM_WORKERS = 32
B_PER_W = B_TOTAL // NUM_WORKERS
CHUNK = 512
N_CHUNKS = B_PER_W // CHUNK
NBUF = 2
NOUT = N_CHUNKS // NBUF

_mesh = plsc.VectorSubcoreMesh(core_axis_name="c", subcore_axis_name="s")

_scratch = (
    [pltpu.VMEM((N_CHUNKS, CHUNK), jnp.int32)]
    + [pltpu.VMEM((CHUNK, EMB), jnp.float32) for _ in range(NBUF)]
    + [pltpu.SemaphoreType.DMA for _ in range(2 * NBUF)]
)


@functools.partial(
    pl.kernel,
    mesh=_mesh,
    out_type=jax.ShapeDtypeStruct((B_TOTAL, EMB), jnp.float32),
    scratch_types=_scratch,
    compiler_params=pltpu.CompilerParams(use_tc_tiling_on_sc=False),
)
def _gather_all(idx_hbm, table_hbm, out_hbm, *scr):
    idx_v = scr[0]
    rows_v = scr[1 : 1 + NBUF]
    gsem = scr[1 + NBUF : 1 + 2 * NBUF]
    wsem = scr[1 + 2 * NBUF : 1 + 3 * NBUF]

    wid = lax.axis_index("s") * 2 + lax.axis_index("c")
    base = wid * B_PER_W

    row0 = pl.multiple_of(wid * N_CHUNKS, N_CHUNKS)
    pltpu.sync_copy(idx_hbm.at[pl.ds(row0, N_CHUNKS)], idx_v)

    def issue_gather(i, b):
        pltpu.async_copy(table_hbm.at[idx_v.at[i]], rows_v[b], gsem[b])

    def wait_gather(i, b):
        pltpu.make_async_copy(table_hbm.at[idx_v.at[i]], rows_v[b], gsem[b]).wait()

    def issue_write(i, b):
        off = pl.multiple_of(base + i * CHUNK, CHUNK)
        pltpu.async_copy(rows_v[b], out_hbm.at[pl.ds(off, CHUNK)], wsem[b])

    def wait_write(b):
        pltpu.make_async_copy(
            rows_v[b], out_hbm.at[pl.ds(base, CHUNK)], wsem[b]
        ).wait()

    for b in range(NBUF):
        issue_gather(b, b)

    def outer(g, _):
        first = g * NBUF
        for b in range(NBUF):
            wait_gather(first + b, b)
            issue_write(first + b, b)
        for b in range(NBUF):
            wait_write(b)
            issue_gather(first + NBUF + b, b)
        return ()

    lax.fori_loop(0, NOUT - 1, outer, ())

    first = (NOUT - 1) * NBUF
    for b in range(NBUF):
        wait_gather(first + b, b)
        issue_write(first + b, b)
    for b in range(NBUF):
        wait_write(b)


def kernel(x, table):
    idx2d = x.reshape(NUM_WORKERS * N_CHUNKS, CHUNK)
    out = _gather_all(idx2d, table)
    return out.reshape(BATCH, SEQ, EMB)


# speedup vs baseline: 1.0054x; 1.0054x over previous
import functools

import jax
import jax.numpy as jnp
from jax import lax
from jax.experimental import pallas as pl
from jax.experimental.pallas import tpu as pltpu
from jax.experimental.pallas import tpu_sc as plsc

EMB = 64
BATCH = 4096
SEQ = 200
NUM_WORKERS = 32
ROWS_PER_W = BATCH // NUM_WORKERS
NBUF = 4
NOUT = ROWS_PER_W // NBUF

_mesh = plsc.VectorSubcoreMesh(core_axis_name="c", subcore_axis_name="s")

_scratch = (
    [pltpu.VMEM((ROWS_PER_W, SEQ), jnp.int32)]
    + [pltpu.VMEM((SEQ, EMB), jnp.float32) for _ in range(NBUF)]
    + [pltpu.SemaphoreType.DMA for _ in range(2 * NBUF)]
)


@functools.partial(
    pl.kernel,
    mesh=_mesh,
    out_type=jax.ShapeDtypeStruct((BATCH, SEQ, EMB), jnp.float32),
    scratch_types=_scratch,
    compiler_params=pltpu.CompilerParams(use_tc_tiling_on_sc=False),
)
def _gather_all(idx_hbm, table_hbm, out_hbm, *scr):
    idx_v = scr[0]
    rows_v = scr[1 : 1 + NBUF]
    gsem = scr[1 + NBUF : 1 + 2 * NBUF]
    wsem = scr[1 + 2 * NBUF : 1 + 3 * NBUF]

    wid = lax.axis_index("s") * 2 + lax.axis_index("c")
    row0 = pl.multiple_of(wid * ROWS_PER_W, ROWS_PER_W)

    pltpu.sync_copy(idx_hbm.at[pl.ds(row0, ROWS_PER_W)], idx_v)

    def issue_gather(i, b):
        pltpu.async_copy(table_hbm.at[idx_v.at[i]], rows_v[b], gsem[b])

    def wait_gather(i, b):
        pltpu.make_async_copy(table_hbm.at[idx_v.at[i]], rows_v[b], gsem[b]).wait()

    def issue_write(i, b):
        pltpu.async_copy(rows_v[b], out_hbm.at[row0 + i], wsem[b])

    def wait_write(b):
        pltpu.make_async_copy(rows_v[b], out_hbm.at[row0], wsem[b]).wait()

    for b in range(NBUF):
        issue_gather(b, b)

    def outer(g, _):
        first = g * NBUF
        for b in range(NBUF):
            wait_gather(first + b, b)
            issue_write(first + b, b)
        for b in range(NBUF):
            wait_write(b)
            issue_gather(first + NBUF + b, b)
        return ()

    lax.fori_loop(0, NOUT - 1, outer, ())

    first = (NOUT - 1) * NBUF
    for b in range(NBUF):
        wait_gather(first + b, b)
        issue_write(first + b, b)
    for b in range(NBUF):
        wait_write(b)


def kernel(x, table):
    return _gather_all(x, table)


# speedup vs baseline: 1.2286x; 1.2220x over previous
import functools

import jax
import jax.numpy as jnp
from jax import lax
from jax.experimental import pallas as pl
from jax.experimental.pallas import tpu as pltpu
from jax.experimental.pallas import tpu_sc as plsc

EMB = 64
PAD = 128
BATCH = 4096
SEQ = 200
B_TOTAL = BATCH * SEQ
NUM_WORKERS = 32
ROWS_PER_W = BATCH // NUM_WORKERS
B_PER_W = B_TOTAL // NUM_WORKERS
NBUF = 4
NOUT = ROWS_PER_W // NBUF

_mesh = plsc.VectorSubcoreMesh(core_axis_name="c", subcore_axis_name="s")

_scratch = (
    [pltpu.VMEM((B_PER_W,), jnp.int32)]
    + [pltpu.VMEM((SEQ, PAD), jnp.float32) for _ in range(NBUF)]
    + [pltpu.SemaphoreType.DMA for _ in range(2 * NBUF)]
)


@functools.partial(
    pl.kernel,
    mesh=_mesh,
    out_type=jax.ShapeDtypeStruct((BATCH, SEQ, PAD), jnp.float32),
    scratch_types=_scratch,
    compiler_params=pltpu.CompilerParams(use_tc_tiling_on_sc=True),
)
def _gather_all(idx_hbm, table_hbm, out_hbm, *scr):
    idx_v = scr[0]
    rows_v = scr[1 : 1 + NBUF]
    gsem = scr[1 + NBUF : 1 + 2 * NBUF]
    wsem = scr[1 + 2 * NBUF : 1 + 3 * NBUF]

    wid = lax.axis_index("s") * 2 + lax.axis_index("c")
    base = pl.multiple_of(wid * B_PER_W, B_PER_W)
    row0 = wid * ROWS_PER_W

    pltpu.sync_copy(idx_hbm.at[pl.ds(base, B_PER_W)], idx_v)

    def gather_ref(i, b):
        src = table_hbm.at[idx_v.at[pl.ds(i * SEQ, SEQ)]]
        return pltpu.make_async_copy(src, rows_v[b], gsem[b])

    def issue_gather(i, b):
        gather_ref(i, b).start()

    def wait_gather(i, b):
        gather_ref(i, b).wait()

    def issue_write(i, b):
        pltpu.async_copy(rows_v[b], out_hbm.at[row0 + i], wsem[b])

    def wait_write(b):
        pltpu.make_async_copy(rows_v[b], out_hbm.at[row0], wsem[b]).wait()

    for b in range(NBUF):
        issue_gather(b, b)

    def outer(g, _):
        first = g * NBUF
        for b in range(NBUF):
            wait_gather(first + b, b)
            issue_write(first + b, b)
        for b in range(NBUF):
            wait_write(b)
            issue_gather(first + NBUF + b, b)
        return ()

    lax.fori_loop(0, NOUT - 1, outer, ())

    first = (NOUT - 1) * NBUF
    for b in range(NBUF):
        wait_gather(first + b, b)
        issue_write(first + b, b)
    for b in range(NBUF):
        wait_write(b)


def kernel(x, table):
    x1 = x.reshape(B_TOTAL)
    tpad = jnp.pad(table, ((0, 0), (0, PAD - EMB)))
    out = _gather_all(x1, tpad)
    return out[:, :, :EMB]
